# baseline (speedup 1.0000x reference)
import jax
import jax.numpy as jnp
from jax import lax
from jax.experimental import pallas as pl
from jax.experimental.pallas import tpu as pltpu
from jax.experimental.pallas import tpu_sc as plsc

_VOCAB = 1000000
_DIM = 32
_BATCH = 16384
_L = 16

_NC = 2
_NS = 16
_NW = _NC * _NS
_BPW = _BATCH // _NW


def _rsqrt16(x):
    i = lax.bitcast_convert_type(x, jnp.int32)
    i = jnp.int32(0x5F3759DF) - (i >> 1)
    y = lax.bitcast_convert_type(i, jnp.float32)
    half = jnp.float32(0.5)
    three_half = jnp.float32(1.5)
    for _ in range(3):
        y = y * (three_half - half * x * y * y)
    return y


def _sc_body(tableT_hbm, key_hbm, traceT_hbm, outT_hbm, idx_v, g_v, tr_v,
             sem_g, sem_t):
    wid = lax.axis_index("s") * _NC + lax.axis_index("c")
    base = wid * _BPW

    pltpu.sync_copy(key_hbm.at[pl.ds(base, _BPW)], idx_v)
    tcopy = pltpu.async_copy(traceT_hbm.at[:, pl.ds(base, _BPW)], tr_v, sem_t)

    batch = 8
    for d0 in range(0, _DIM, batch):
        descs = [
            pltpu.async_copy(tableT_hbm.at[d].at[idx_v], g_v.at[d], sem_g)
            for d in range(d0, d0 + batch)
        ]
        for desc in descs:
            desc.wait()
    tcopy.wait()

    def chunk(c, _):
        off = pl.multiple_of(c * _L, _L)
        acc = jnp.full((_L,), 1e-30, jnp.float32)
        vs = []
        for d in range(_DIM):
            v = g_v[d, pl.ds(off, _L)] + tr_v[d, pl.ds(off, _L)]
            acc = acc + v * v
            vs.append(v)
        rs = _rsqrt16(acc)
        for d in range(_DIM):
            g_v[d, pl.ds(off, _L)] = vs[d] * rs
        return _

    lax.fori_loop(0, _BPW // _L, chunk, None)
    pltpu.sync_copy(g_v, outT_hbm.at[:, pl.ds(base, _BPW)])


@jax.jit
def _sc_call(table, key, trace):
    mesh = plsc.VectorSubcoreMesh(core_axis_name="c", subcore_axis_name="s")
    f = pl.kernel(
        _sc_body,
        out_type=jax.ShapeDtypeStruct((_DIM, _BATCH), jnp.float32),
        mesh=mesh,
        scratch_types=[
            pltpu.VMEM((_BPW,), jnp.int32),
            pltpu.VMEM((_DIM, _BPW), jnp.float32),
            pltpu.VMEM((_DIM, _BPW), jnp.float32),
            pltpu.SemaphoreType.DMA,
            pltpu.SemaphoreType.DMA,
        ],
        compiler_params=pltpu.CompilerParams(use_tc_tiling_on_sc=False),
    )
    return f(table.T, key, trace.T).T


def kernel(arg0_unused, trace, arg2_unused, key, table):
    return _sc_call(table, key.astype(jnp.int32), trace)


# speedup vs baseline: 3.2211x; 3.2211x over previous
import jax
import jax.numpy as jnp
from jax import lax
from jax.experimental import pallas as pl
from jax.experimental.pallas import tpu as pltpu
from jax.experimental.pallas import tpu_sc as plsc

_VOCAB = 1000000
_DIM = 32
_BATCH = 16384
_L = 16

_NC = 2
_NS = 16
_NW = _NC * _NS
_BPW = _BATCH // _NW

_KBLK = 2048


def _detile_body(tt_ref, out_ref):
    x = tt_ref[...]
    eye = jax.lax.broadcasted_iota(jnp.int32, (_DIM, _DIM), 0) == \
        jax.lax.broadcasted_iota(jnp.int32, (_DIM, _DIM), 1)
    y = jax.lax.dot_general(x, eye.astype(jnp.float32), (((0,), (0,)), ((), ())),
                            precision=jax.lax.Precision.HIGHEST,
                            preferred_element_type=jnp.float32)
    y3 = y.reshape(_KBLK // 4, 4, _DIM)
    parts = [y3[:, c, :] for c in range(4)]
    out_ref[...] = jnp.concatenate(parts, axis=1)


def _tc_linearize(tableT):
    grid = (_VOCAB + _KBLK - 1) // _KBLK
    return pl.pallas_call(
        _detile_body,
        grid=(grid,),
        in_specs=[pl.BlockSpec((_DIM, _KBLK), lambda i: (0, i))],
        out_specs=pl.BlockSpec((_KBLK // 4, 128), lambda i: (i, 0)),
        out_shape=jax.ShapeDtypeStruct((_VOCAB * _DIM // 128, 128), jnp.float32),
    )(tableT)


def _rsqrt16(x):
    i = lax.bitcast_convert_type(x, jnp.int32)
    i = jnp.int32(0x5F3759DF) - (i >> 1)
    y = lax.bitcast_convert_type(i, jnp.float32)
    half = jnp.float32(0.5)
    three_half = jnp.float32(1.5)
    for _ in range(3):
        y = y * (three_half - half * x * y * y)
    return y


_GDN = lax.GatherDimensionNumbers(
    offset_dims=(), collapsed_slice_dims=(0,), start_index_map=(0,))


def _shuffle(v, perm):
    return lax.gather(v, perm[:, None], _GDN, slice_sizes=(1,),
                      mode=lax.GatherScatterMode.PROMISE_IN_BOUNDS)


def _sc_body(table_hbm, key_hbm, trace_hbm, out_hbm, idx_v, rows_v, tr_v,
             sem_g, sem_t):
    wid = lax.axis_index("s") * _NC + lax.axis_index("c")
    base = wid * _BPW

    pltpu.sync_copy(key_hbm.at[pl.ds(base, _BPW)], idx_v)
    gather = pltpu.async_copy(table_hbm.at[idx_v], rows_v, sem_g)
    tcopy = pltpu.async_copy(trace_hbm.at[pl.ds(base, _BPW)], tr_v, sem_t)
    gather.wait()
    tcopy.wait()

    def row(r, _):
        v0 = rows_v[r, 0:16] + tr_v[r, 0:16]
        v1 = rows_v[r, 16:32] + tr_v[r, 16:32]
        s = v0 * v0 + v1 * v1
        for k in (8, 4, 2, 1):
            perm = lax.iota(jnp.int32, _L) ^ k
            s = s + _shuffle(s, perm)
        rs = _rsqrt16(s + jnp.float32(1e-30))
        rows_v[r, 0:16] = v0 * rs
        rows_v[r, 16:32] = v1 * rs
        return _

    lax.fori_loop(0, _BPW, row, None)
    pltpu.sync_copy(rows_v, out_hbm.at[pl.ds(base, _BPW)])


def _sc_gather_norm(table_lin, key, trace):
    mesh = plsc.VectorSubcoreMesh(core_axis_name="c", subcore_axis_name="s")
    f = pl.kernel(
        _sc_body,
        out_type=jax.ShapeDtypeStruct((_BATCH, _DIM), jnp.float32),
        mesh=mesh,
        scratch_types=[
            pltpu.VMEM((_BPW,), jnp.int32),
            pltpu.VMEM((_BPW, _DIM), jnp.float32),
            pltpu.VMEM((_BPW, _DIM), jnp.float32),
            pltpu.SemaphoreType.DMA,
            pltpu.SemaphoreType.DMA,
        ],
        compiler_params=pltpu.CompilerParams(use_tc_tiling_on_sc=False),
    )
    return f(table_lin, key, trace)


@jax.jit
def _impl(table, key, trace):
    lin = _tc_linearize(table.T)
    table_lin = lin.reshape(_VOCAB, _DIM)
    return _sc_gather_norm(table_lin, key, trace)


def kernel(arg0_unused, trace, arg2_unused, key, table):
    return _impl(table, key.astype(jnp.int32), trace)


# speedup vs baseline: 15.3042x; 4.7512x over previous
import jax
import jax.numpy as jnp
from jax import lax
from jax.experimental import pallas as pl
from jax.experimental.pallas import tpu as pltpu
from jax.experimental.pallas import tpu_sc as plsc

_VOCAB = 1000000
_DIM = 32
_BATCH = 16384
_L = 16

_NC = 2
_NS = 16
_NW = _NC * _NS
_BPW = _BATCH // _NW

_S = 1 << 18
_KB = 4096
_GRID = _S // _KB


def _detile_body(t0, t1, t2, t3, out_ref):
    x = jnp.concatenate([t0[...], t1[...], t2[...], t3[...]], axis=0)
    out_ref[...] = jnp.transpose(x)


def _tc_linearize(tableT):
    last = (_VOCAB - 1) // _KB
    in_specs = [
        pl.BlockSpec(
            (_DIM, _KB),
            lambda i, c=c: (0, jnp.minimum((c * _S) // _KB + i, last)))
        for c in range(4)
    ]
    return pl.pallas_call(
        _detile_body,
        grid=(_GRID,),
        in_specs=in_specs,
        out_specs=pl.BlockSpec((_KB, 128), lambda i: (i, 0)),
        out_shape=jax.ShapeDtypeStruct((_S, 128), jnp.float32),
    )(tableT, tableT, tableT, tableT)


def _rsqrt16(x):
    i = lax.bitcast_convert_type(x, jnp.int32)
    i = jnp.int32(0x5F3759DF) - (i >> 1)
    y = lax.bitcast_convert_type(i, jnp.float32)
    half = jnp.float32(0.5)
    three_half = jnp.float32(1.5)
    for _ in range(3):
        y = y * (three_half - half * x * y * y)
    return y


_GDN = lax.GatherDimensionNumbers(
    offset_dims=(), collapsed_slice_dims=(0,), start_index_map=(0,))


def _shuffle(v, perm):
    return lax.gather(v, perm[:, None], _GDN, slice_sizes=(1,),
                      mode=lax.GatherScatterMode.PROMISE_IN_BOUNDS)


def _sc_body(table_hbm, key_hbm, trace_hbm, out_hbm, idx_v, a_v, c32_v,
             rows_v, tr_v, out_v, sem_g, sem_t):
    wid = lax.axis_index("s") * _NC + lax.axis_index("c")
    base = wid * _BPW

    pltpu.sync_copy(key_hbm.at[pl.ds(base, _BPW)], idx_v)
    tcopy = pltpu.async_copy(trace_hbm.at[pl.ds(base, _BPW)], tr_v, sem_t)

    def split(g, _):
        goff = pl.multiple_of(g * _L, _L)
        ks = idx_v[pl.ds(goff, _L)]
        a_v[pl.ds(goff, _L)] = ks & jnp.int32(_S - 1)
        c32_v[pl.ds(goff, _L)] = (ks >> 18) << 5
        return _

    lax.fori_loop(0, _BPW // _L, split, None)

    gather = pltpu.async_copy(table_hbm.at[a_v], rows_v, sem_g)
    gather.wait()
    tcopy.wait()

    def group(g, _):
        goff = pl.multiple_of(g * _L, _L)
        cvec = c32_v[pl.ds(goff, _L)]
        for i in range(_L):
            r = goff + i
            off = cvec[i]
            v0 = rows_v[r, pl.ds(off, _L)] + tr_v[r, 0:16]
            v1 = rows_v[r, pl.ds(off + _L, _L)] + tr_v[r, 16:32]
            s = v0 * v0 + v1 * v1
            for k in (8, 4, 2, 1):
                perm = lax.iota(jnp.int32, _L) ^ k
                s = s + _shuffle(s, perm)
            rs = _rsqrt16(s + jnp.float32(1e-30))
            out_v[r, 0:16] = v0 * rs
            out_v[r, 16:32] = v1 * rs
        return _

    lax.fori_loop(0, _BPW // _L, group, None)
    pltpu.sync_copy(out_v, out_hbm.at[pl.ds(base, _BPW)])


def _sc_gather_norm(table_lin, key, trace):
    mesh = plsc.VectorSubcoreMesh(core_axis_name="c", subcore_axis_name="s")
    f = pl.kernel(
        _sc_body,
        out_type=jax.ShapeDtypeStruct((_BATCH, _DIM), jnp.float32),
        mesh=mesh,
        scratch_types=[
            pltpu.VMEM((_BPW,), jnp.int32),
            pltpu.VMEM((_BPW,), jnp.int32),
            pltpu.VMEM((_BPW,), jnp.int32),
            pltpu.VMEM((_BPW, 128), jnp.float32),
            pltpu.VMEM((_BPW, _DIM), jnp.float32),
            pltpu.VMEM((_BPW, _DIM), jnp.float32),
            pltpu.SemaphoreType.DMA,
            pltpu.SemaphoreType.DMA,
        ],
        compiler_params=pltpu.CompilerParams(use_tc_tiling_on_sc=False),
    )
    return f(table_lin, key, trace)


@jax.jit
def _impl(table, key, trace):
    lin = _tc_linearize(table.T)
    table_lin = lin.reshape(_S, 128)
    return _sc_gather_norm(table_lin, key, trace)


def kernel(arg0_unused, trace, arg2_unused, key, table):
    return _impl(table, key.astype(jnp.int32), trace)


# speedup vs baseline: 16.7822x; 1.0966x over previous
import jax
import jax.numpy as jnp
from jax import lax
from jax.experimental import pallas as pl
from jax.experimental.pallas import tpu as pltpu
from jax.experimental.pallas import tpu_sc as plsc

_VOCAB = 1000000
_DIM = 32
_BATCH = 16384
_L = 16

_NC = 2
_NS = 16
_NW = _NC * _NS
_BPW = _BATCH // _NW

_S = 1 << 18
_KB = 8192
_GRID = _S // _KB


def _detile_body(t0, t1, t2, t3, out_ref):
    x = jnp.concatenate([t0[...], t1[...], t2[...], t3[...]], axis=0)
    out_ref[...] = jnp.transpose(x)


def _tc_linearize(tableT):
    last = (_VOCAB - 1) // _KB
    in_specs = [
        pl.BlockSpec(
            (_DIM, _KB),
            lambda i, c=c: (0, jnp.minimum((c * _S) // _KB + i, last)))
        for c in range(4)
    ]
    return pl.pallas_call(
        _detile_body,
        grid=(_GRID,),
        in_specs=in_specs,
        out_specs=pl.BlockSpec((_KB, 128), lambda i: (i, 0)),
        out_shape=jax.ShapeDtypeStruct((_S, 128), jnp.float32),
    )(tableT, tableT, tableT, tableT)


def _rsqrt16(x):
    i = lax.bitcast_convert_type(x, jnp.int32)
    i = jnp.int32(0x5F3759DF) - (i >> 1)
    y = lax.bitcast_convert_type(i, jnp.float32)
    half = jnp.float32(0.5)
    three_half = jnp.float32(1.5)
    for _ in range(3):
        y = y * (three_half - half * x * y * y)
    return y


_GDN = lax.GatherDimensionNumbers(
    offset_dims=(), collapsed_slice_dims=(0,), start_index_map=(0,))


def _shuffle(v, perm):
    return lax.gather(v, perm[:, None], _GDN, slice_sizes=(1,),
                      mode=lax.GatherScatterMode.PROMISE_IN_BOUNDS)


def _sc_body(table_hbm, key_hbm, trace_hbm, out_hbm, idx_v, a_v, c32_v,
             rows_v, tr_v, out_v, sem_g, sem_t):
    wid = lax.axis_index("s") * _NC + lax.axis_index("c")
    base = wid * _BPW

    pltpu.sync_copy(key_hbm.at[pl.ds(base, _BPW)], idx_v)
    tcopy = pltpu.async_copy(trace_hbm.at[pl.ds(base, _BPW)], tr_v, sem_t)

    def split(g, _):
        goff = pl.multiple_of(g * _L, _L)
        ks = idx_v[pl.ds(goff, _L)]
        a_v[pl.ds(goff, _L)] = ks & jnp.int32(_S - 1)
        c32_v[pl.ds(goff, _L)] = (ks >> 18) << 5
        return _

    lax.fori_loop(0, _BPW // _L, split, None)

    gather = pltpu.async_copy(table_hbm.at[a_v], rows_v, sem_g)
    gather.wait()
    tcopy.wait()

    def group(g, _):
        goff = pl.multiple_of(g * _L, _L)
        cvec = c32_v[pl.ds(goff, _L)]
        for i in range(_L):
            r = goff + i
            off = cvec[i]
            v0 = rows_v[r, pl.ds(off, _L)] + tr_v[r, 0:16]
            v1 = rows_v[r, pl.ds(off + _L, _L)] + tr_v[r, 16:32]
            s = v0 * v0 + v1 * v1
            for k in (8, 4, 2, 1):
                perm = lax.iota(jnp.int32, _L) ^ k
                s = s + _shuffle(s, perm)
            rs = _rsqrt16(s + jnp.float32(1e-30))
            out_v[r, 0:16] = v0 * rs
            out_v[r, 16:32] = v1 * rs
        return _

    lax.fori_loop(0, _BPW // _L, group, None)
    pltpu.sync_copy(out_v, out_hbm.at[pl.ds(base, _BPW)])


def _sc_gather_norm(table_lin, key, trace):
    mesh = plsc.VectorSubcoreMesh(core_axis_name="c", subcore_axis_name="s")
    f = pl.kernel(
        _sc_body,
        out_type=jax.ShapeDtypeStruct((_BATCH, _DIM), jnp.float32),
        mesh=mesh,
        scratch_types=[
            pltpu.VMEM((_BPW,), jnp.int32),
            pltpu.VMEM((_BPW,), jnp.int32),
            pltpu.VMEM((_BPW,), jnp.int32),
            pltpu.VMEM((_BPW, 128), jnp.float32),
            pltpu.VMEM((_BPW, _DIM), jnp.float32),
            pltpu.VMEM((_BPW, _DIM), jnp.float32),
            pltpu.SemaphoreType.DMA,
            pltpu.SemaphoreType.DMA,
        ],
        compiler_params=pltpu.CompilerParams(use_tc_tiling_on_sc=False),
    )
    return f(table_lin, key, trace)


@jax.jit
def _impl(table, key, trace):
    lin = _tc_linearize(table.T)
    table_lin = lin.reshape(_S, 128)
    return _sc_gather_norm(table_lin, key, trace)


def kernel(arg0_unused, trace, arg2_unused, key, table):
    return _impl(table, key.astype(jnp.int32), trace)


# speedup vs baseline: 17.0777x; 1.0176x over previous
import jax
import jax.numpy as jnp
from jax import lax
from jax.experimental import pallas as pl
from jax.experimental.pallas import tpu as pltpu
from jax.experimental.pallas import tpu_sc as plsc

_VOCAB = 1000000
_DIM = 32
_BATCH = 16384
_L = 16

_NC = 2
_NS = 16
_NW = _NC * _NS
_BPW = _BATCH // _NW

_S = 1 << 18
_KB = 16384
_GRID = _S // _KB


def _detile_body(t0, t1, t2, t3, out_ref):
    x = jnp.concatenate([t0[...], t1[...], t2[...], t3[...]], axis=0)
    out_ref[...] = jnp.transpose(x)


def _tc_linearize(tableT):
    last = (_VOCAB - 1) // _KB
    in_specs = [
        pl.BlockSpec(
            (_DIM, _KB),
            lambda i, c=c: (0, jnp.minimum((c * _S) // _KB + i, last)))
        for c in range(4)
    ]
    return pl.pallas_call(
        _detile_body,
        grid=(_GRID,),
        in_specs=in_specs,
        out_specs=pl.BlockSpec((_KB, 128), lambda i: (i, 0)),
        out_shape=jax.ShapeDtypeStruct((_S, 128), jnp.float32),
    )(tableT, tableT, tableT, tableT)


def _rsqrt16(x):
    i = lax.bitcast_convert_type(x, jnp.int32)
    i = jnp.int32(0x5F3759DF) - (i >> 1)
    y = lax.bitcast_convert_type(i, jnp.float32)
    half = jnp.float32(0.5)
    three_half = jnp.float32(1.5)
    for _ in range(3):
        y = y * (three_half - half * x * y * y)
    return y


_GDN = lax.GatherDimensionNumbers(
    offset_dims=(), collapsed_slice_dims=(0,), start_index_map=(0,))


def _shuffle(v, perm):
    return lax.gather(v, perm[:, None], _GDN, slice_sizes=(1,),
                      mode=lax.GatherScatterMode.PROMISE_IN_BOUNDS)


def _sc_body(table_hbm, key_hbm, trace_hbm, out_hbm, idx_v, a_v, c32_v,
             rows_v, tr_v, out_v, sem_g, sem_t):
    wid = lax.axis_index("s") * _NC + lax.axis_index("c")
    base = wid * _BPW

    pltpu.sync_copy(key_hbm.at[pl.ds(base, _BPW)], idx_v)
    tcopy = pltpu.async_copy(trace_hbm.at[pl.ds(base, _BPW)], tr_v, sem_t)

    def split(g, _):
        goff = pl.multiple_of(g * _L, _L)
        ks = idx_v[pl.ds(goff, _L)]
        a_v[pl.ds(goff, _L)] = ks & jnp.int32(_S - 1)
        c32_v[pl.ds(goff, _L)] = (ks >> 18) << 5
        return _

    lax.fori_loop(0, _BPW // _L, split, None)

    gather = pltpu.async_copy(table_hbm.at[a_v], rows_v, sem_g)
    gather.wait()
    tcopy.wait()

    def group(g, _):
        goff = pl.multiple_of(g * _L, _L)
        cvec = c32_v[pl.ds(goff, _L)]
        for i in range(_L):
            r = goff + i
            off = cvec[i]
            v0 = rows_v[r, pl.ds(off, _L)] + tr_v[r, 0:16]
            v1 = rows_v[r, pl.ds(off + _L, _L)] + tr_v[r, 16:32]
            s = v0 * v0 + v1 * v1
            for k in (8, 4, 2, 1):
                perm = lax.iota(jnp.int32, _L) ^ k
                s = s + _shuffle(s, perm)
            rs = _rsqrt16(s + jnp.float32(1e-30))
            out_v[r, 0:16] = v0 * rs
            out_v[r, 16:32] = v1 * rs
        return _

    lax.fori_loop(0, _BPW // _L, group, None)
    pltpu.sync_copy(out_v, out_hbm.at[pl.ds(base, _BPW)])


def _sc_gather_norm(table_lin, key, trace):
    mesh = plsc.VectorSubcoreMesh(core_axis_name="c", subcore_axis_name="s")
    f = pl.kernel(
        _sc_body,
        out_type=jax.ShapeDtypeStruct((_BATCH, _DIM), jnp.float32),
        mesh=mesh,
        scratch_types=[
            pltpu.VMEM((_BPW,), jnp.int32),
            pltpu.VMEM((_BPW,), jnp.int32),
            pltpu.VMEM((_BPW,), jnp.int32),
            pltpu.VMEM((_BPW, 128), jnp.float32),
            pltpu.VMEM((_BPW, _DIM), jnp.float32),
            pltpu.VMEM((_BPW, _DIM), jnp.float32),
            pltpu.SemaphoreType.DMA,
            pltpu.SemaphoreType.DMA,
        ],
        compiler_params=pltpu.CompilerParams(use_tc_tiling_on_sc=False),
    )
    return f(table_lin, key, trace)


@jax.jit
def _impl(table, key, trace):
    lin = _tc_linearize(table.T)
    table_lin = lin.reshape(_S, 128)
    return _sc_gather_norm(table_lin, key, trace)


def kernel(arg0_unused, trace, arg2_unused, key, table):
    return _impl(table, key.astype(jnp.int32), trace)
